# baseline (speedup 1.0000x reference)
import functools

import jax
import jax.numpy as jnp
from jax import lax
from jax.experimental import pallas as pl
from jax.experimental.pallas import tpu as pltpu
from jax.experimental.pallas import tpu_sc as plsc

N, E, D, R, B_ = 10000, 320000, 128, 8, 8
NC, NS = 2, 16
NW = NC * NS
CHUNK = 128
CPW = 80
HHALF = CPW // 2
NHALF = CPW // HHALF
EPAD = NW * CPW * CHUNK
NPAD = 10240
ROWS_PT = NPAD // NS
BN_EPS = 1e-5
NBLK = 2000
NB = N // NBLK


def _wcomb_body(coeff_ref, basis_ref, w_ref):
    w_ref[...] = jnp.dot(coeff_ref[...], basis_ref[...],
                         preferred_element_type=jnp.float32)


def _wcat(coeff, basis):
    w = pl.pallas_call(
        _wcomb_body,
        out_shape=jax.ShapeDtypeStruct((R, D * D), jnp.float32),
    )(coeff, basis.reshape(B_, D * D))
    return w.reshape(R, D, D).transpose(1, 0, 2).reshape(D, R * D)


def _hall_body(x_ref, wcat_ref, out_ref):
    out_ref[...] = jnp.dot(x_ref[...], wcat_ref[...],
                           preferred_element_type=jnp.float32)


def _hall(x, wcat):
    return pl.pallas_call(
        _hall_body,
        grid=(NB,),
        in_specs=[
            pl.BlockSpec((NBLK, D), lambda i: (i, 0)),
            pl.BlockSpec((D, R * D), lambda i: (0, 0)),
        ],
        out_specs=pl.BlockSpec((NBLK, R * D), lambda i: (i, 0)),
        out_shape=jax.ShapeDtypeStruct((N, R * D), jnp.float32),
    )(x, wcat)


def _hall_bn_body(y_ref, st_ref, g_ref, gb_ref, wcat_ref, out_ref, x_ref):
    mean = st_ref[0] * (1.0 / N)
    var = st_ref[1] * (1.0 / N) - mean * mean
    scale = lax.rsqrt(var + BN_EPS) * g_ref[...]
    shift = gb_ref[...] - mean * scale
    xb = jnp.maximum(y_ref[...] * scale + shift, 0.0)
    x_ref[...] = xb
    out_ref[...] = jnp.dot(xb, wcat_ref[...],
                           preferred_element_type=jnp.float32)


def _hall_bn(y, stats, bng, bnb, wcat):
    return pl.pallas_call(
        _hall_bn_body,
        grid=(NB,),
        in_specs=[
            pl.BlockSpec((NBLK, D), lambda i: (i, 0)),
            pl.BlockSpec((2, D), lambda i: (0, 0)),
            pl.BlockSpec((D,), lambda i: (0,)),
            pl.BlockSpec((D,), lambda i: (0,)),
            pl.BlockSpec((D, R * D), lambda i: (0, 0)),
        ],
        out_specs=[
            pl.BlockSpec((NBLK, R * D), lambda i: (i, 0)),
            pl.BlockSpec((NBLK, D), lambda i: (i, 0)),
        ],
        out_shape=[
            jax.ShapeDtypeStruct((N, R * D), jnp.float32),
            jax.ShapeDtypeStruct((N, D), jnp.float32),
        ],
    )(y, stats, bng, bnb, wcat)


def _stage1_body(p_ref, cb_ref, w1_ref, b1_ref, hdn_ref, st_ref):
    i = pl.program_id(0)
    agg = p_ref[0] + p_ref[1] + cb_ref[...]
    hdn = jnp.dot(agg, w1_ref[...], preferred_element_type=jnp.float32)
    hdn = hdn + b1_ref[...]
    hdn_ref[...] = hdn
    s0 = jnp.sum(hdn, axis=0, keepdims=True)
    s1 = jnp.sum(hdn * hdn, axis=0, keepdims=True)
    st = jnp.concatenate([s0, s1], axis=0)

    @pl.when(i == 0)
    def _():
        st_ref[...] = jnp.zeros_like(st_ref)

    st_ref[...] += st


def _stage1(partials, cb, w1, b1):
    return pl.pallas_call(
        _stage1_body,
        grid=(NB,),
        in_specs=[
            pl.BlockSpec((NC, NBLK, D), lambda i: (0, i, 0)),
            pl.BlockSpec((D,), lambda i: (0,)),
            pl.BlockSpec((D, D), lambda i: (0, 0)),
            pl.BlockSpec((D,), lambda i: (0,)),
        ],
        out_specs=[
            pl.BlockSpec((NBLK, D), lambda i: (i, 0)),
            pl.BlockSpec((2, D), lambda i: (0, 0)),
        ],
        out_shape=[
            jax.ShapeDtypeStruct((N, D), jnp.float32),
            jax.ShapeDtypeStruct((2, D), jnp.float32),
        ],
    )(partials, cb, w1, b1)


def _stage2_body(hdn_ref, st_ref, g_ref, gb_ref, w2_ref, b2_ref,
                 y_ref, st2_ref):
    i = pl.program_id(0)
    mean = st_ref[0] * (1.0 / N)
    var = st_ref[1] * (1.0 / N) - mean * mean
    scale = lax.rsqrt(var + BN_EPS) * g_ref[...]
    shift = gb_ref[...] - mean * scale
    xb = jnp.maximum(hdn_ref[...] * scale + shift, 0.0)
    y = jnp.dot(xb, w2_ref[...], preferred_element_type=jnp.float32)
    y = y + b2_ref[...]
    y_ref[...] = y
    s0 = jnp.sum(y, axis=0, keepdims=True)
    s1 = jnp.sum(y * y, axis=0, keepdims=True)
    st = jnp.concatenate([s0, s1], axis=0)

    @pl.when(i == 0)
    def _():
        st2_ref[...] = jnp.zeros_like(st2_ref)

    st2_ref[...] += st


def _stage2(hdn, stats, g1, gb1, w2, b2):
    return pl.pallas_call(
        _stage2_body,
        grid=(NB,),
        in_specs=[
            pl.BlockSpec((NBLK, D), lambda i: (i, 0)),
            pl.BlockSpec((2, D), lambda i: (0, 0)),
            pl.BlockSpec((D,), lambda i: (0,)),
            pl.BlockSpec((D,), lambda i: (0,)),
            pl.BlockSpec((D, D), lambda i: (0, 0)),
            pl.BlockSpec((D,), lambda i: (0,)),
        ],
        out_specs=[
            pl.BlockSpec((NBLK, D), lambda i: (i, 0)),
            pl.BlockSpec((2, D), lambda i: (0, 0)),
        ],
        out_shape=[
            jax.ShapeDtypeStruct((N, D), jnp.float32),
            jax.ShapeDtypeStruct((2, D), jnp.float32),
        ],
    )(hdn, stats, g1, gb1, w2, b2)


def _stage3_body(y_ref, st_ref, g_ref, gb_ref, x_ref):
    mean = st_ref[0] * (1.0 / N)
    var = st_ref[1] * (1.0 / N) - mean * mean
    scale = lax.rsqrt(var + BN_EPS) * g_ref[...]
    shift = gb_ref[...] - mean * scale
    x_ref[...] = jnp.maximum(y_ref[...] * scale + shift, 0.0)


def _stage3(y, stats, bng, bnb):
    return pl.pallas_call(
        _stage3_body,
        grid=(NB,),
        in_specs=[
            pl.BlockSpec((NBLK, D), lambda i: (i, 0)),
            pl.BlockSpec((2, D), lambda i: (0, 0)),
            pl.BlockSpec((D,), lambda i: (0,)),
            pl.BlockSpec((D,), lambda i: (0,)),
        ],
        out_specs=pl.BlockSpec((NBLK, D), lambda i: (i, 0)),
        out_shape=jax.ShapeDtypeStruct((N, D), jnp.float32),
    )(y, stats, bng, bnb)


def _edge_agg_body(hall_ref, gidx_ref, dst_ref, out_ref,
                   gidx_v, dst_v, rows0, rows1, sem0, sem1, agg):
    c = lax.axis_index("c")
    s = lax.axis_index("s")
    wid = s * NC + c
    z = jnp.zeros((16,), jnp.float32)

    def zrow(j, carry):
        for k in range(D // 16):
            rows0[j, pl.ds(k * 16, 16)] = z
        return carry

    lax.fori_loop(0, CHUNK, zrow, 0)
    for t in range(ROWS_PT // CHUNK):
        pltpu.sync_copy(rows0, agg.at[pl.ds(s * ROWS_PT + t * CHUNK, CHUNK)])
    plsc.subcore_barrier()

    for p in range(NHALF):
        pltpu.sync_copy(gidx_ref.at[wid * NHALF + p], gidx_v)
        pltpu.sync_copy(dst_ref.at[wid * NHALF + p], dst_v)
        pltpu.async_copy(hall_ref.at[gidx_v.at[0]], rows0, sem0)

        def pair(i, carry):
            j0 = 2 * i
            pltpu.async_copy(hall_ref.at[gidx_v.at[j0 + 1]], rows1, sem1)
            pltpu.make_async_copy(hall_ref.at[gidx_v.at[j0]], rows0,
                                  sem0).wait()
            pltpu.sync_copy(rows0, agg.at[dst_v.at[j0]], add=True)

            @pl.when(j0 + 2 < HHALF)
            def _():
                pltpu.async_copy(hall_ref.at[gidx_v.at[j0 + 2]], rows0, sem0)

            pltpu.make_async_copy(hall_ref.at[gidx_v.at[j0 + 1]], rows1,
                                  sem1).wait()
            pltpu.sync_copy(rows1, agg.at[dst_v.at[j0 + 1]], add=True)
            return carry

        lax.fori_loop(0, HHALF // 2, pair, 0)
    plsc.subcore_barrier()
    pltpu.sync_copy(agg.at[pl.ds(s * ROWS_PT, ROWS_PT)],
                    out_ref.at[c, pl.ds(s * ROWS_PT, ROWS_PT)])


@functools.lru_cache(maxsize=None)
def _build_edge_agg():
    mesh = plsc.VectorSubcoreMesh(core_axis_name="c", subcore_axis_name="s")
    return pl.kernel(
        _edge_agg_body,
        out_type=jax.ShapeDtypeStruct((NC, NPAD, D), jnp.float32),
        name="edge_agg",
        mesh=mesh,
        scratch_types=[
            pltpu.VMEM((HHALF, CHUNK), jnp.int32),
            pltpu.VMEM((HHALF, CHUNK), jnp.int32),
            pltpu.VMEM((CHUNK, D), jnp.float32),
            pltpu.VMEM((CHUNK, D), jnp.float32),
            pltpu.SemaphoreType.DMA,
            pltpu.SemaphoreType.DMA,
            pltpu.VMEM_SHARED((NPAD, D), jnp.float32),
        ],
    )


def _mlp(hall, gidx, dst, cb, w1, b1, g1, gb1, w2, b2):
    partials = _build_edge_agg()(hall.reshape(N * R, D), gidx, dst)
    hdn, st1 = _stage1(partials, cb, w1, b1)
    return _stage2(hdn, st1, g1, gb1, w2, b2)


def kernel(h, edge_index, etypes,
           basis0, coeff0, cb0, w1_0, b1_0, g1_0, gb1_0, w2_0, b2_0,
           bng_0, bnb_0,
           basis1, coeff1, cb1, w1_1, b1_1, g1_1, gb1_1, w2_1, b2_1,
           bng_1, bnb_1):
    src, dst = edge_index[0], edge_index[1]
    gidx = src * R + etypes
    epw = E // NW
    padw = CPW * CHUNK - epw
    pad_g = jnp.tile(jnp.arange(padw, dtype=jnp.int32) * 331 % (R * N),
                     (NW, 1))
    pad_d = jnp.tile(N + jnp.arange(padw, dtype=jnp.int32) % (NPAD - N),
                     (NW, 1))
    gidx = jnp.concatenate([gidx.reshape(NW, epw), pad_g], axis=1)
    dstp = jnp.concatenate([dst.reshape(NW, epw), pad_d], axis=1)
    gidx = gidx.reshape(NW * NHALF, HHALF, CHUNK)
    dstp = dstp.reshape(NW * NHALF, HHALF, CHUNK)

    hall1 = _hall(h, _wcat(coeff0, basis0))
    y1, sty1 = _mlp(hall1, gidx, dstp, cb0, w1_0, b1_0,
                    g1_0, gb1_0, w2_0, b2_0)
    hall2, x1 = _hall_bn(y1, sty1, bng_0, bnb_0, _wcat(coeff1, basis1))
    y2, sty2 = _mlp(hall2, gidx, dstp, cb1, w1_1, b1_1,
                    g1_1, gb1_1, w2_1, b2_1)
    x2 = _stage3(y2, sty2, bng_1, bnb_1)
    return jnp.stack([h, x1, x2])


# speedup vs baseline: 1.2398x; 1.2398x over previous
import functools

import jax
import jax.numpy as jnp
from jax import lax
from jax.experimental import pallas as pl
from jax.experimental.pallas import tpu as pltpu
from jax.experimental.pallas import tpu_sc as plsc

N, E, D, R, B_ = 10000, 320000, 128, 8, 8
NC, NS = 2, 16
NW = NC * NS
CHUNK = 128
CPW = 80
HHALF = CPW // 2
NHALF = CPW // HHALF
EPAD = NW * CPW * CHUNK
NPAD = 10240
ROWS_PT = NPAD // NS
BN_EPS = 1e-5
NBLK = 2000
NB = N // NBLK


def _wcomb_body(coeff_ref, basis_ref, w_ref):
    w_ref[...] = jnp.dot(coeff_ref[...], basis_ref[...],
                         preferred_element_type=jnp.float32)


def _wcomb(coeff, basis):
    w = pl.pallas_call(
        _wcomb_body,
        out_shape=jax.ShapeDtypeStruct((R, D * D), jnp.float32),
    )(coeff, basis.reshape(B_, D * D))
    return w.reshape(R, D, D)


def _hall_body(x_ref, w_ref, out_ref):
    x = x_ref[...]
    for r in range(R):
        out_ref[r] = jnp.dot(x, w_ref[r], preferred_element_type=jnp.float32)


def _hall(x, w):
    return pl.pallas_call(
        _hall_body,
        grid=(NB,),
        in_specs=[
            pl.BlockSpec((NBLK, D), lambda i: (i, 0)),
            pl.BlockSpec((R, D, D), lambda i: (0, 0, 0)),
        ],
        out_specs=pl.BlockSpec((R, NBLK, D), lambda i: (0, i, 0)),
        out_shape=jax.ShapeDtypeStruct((R, N, D), jnp.float32),
    )(x, w)


def _hall_bn_body(y_ref, st_ref, g_ref, gb_ref, w_ref, out_ref, x_ref):
    mean = st_ref[0] * (1.0 / N)
    var = st_ref[1] * (1.0 / N) - mean * mean
    scale = lax.rsqrt(var + BN_EPS) * g_ref[...]
    shift = gb_ref[...] - mean * scale
    xb = jnp.maximum(y_ref[...] * scale + shift, 0.0)
    x_ref[...] = xb
    for r in range(R):
        out_ref[r] = jnp.dot(xb, w_ref[r], preferred_element_type=jnp.float32)


def _hall_bn(y, stats, bng, bnb, w):
    return pl.pallas_call(
        _hall_bn_body,
        grid=(NB,),
        in_specs=[
            pl.BlockSpec((NBLK, D), lambda i: (i, 0)),
            pl.BlockSpec((2, D), lambda i: (0, 0)),
            pl.BlockSpec((D,), lambda i: (0,)),
            pl.BlockSpec((D,), lambda i: (0,)),
            pl.BlockSpec((R, D, D), lambda i: (0, 0, 0)),
        ],
        out_specs=[
            pl.BlockSpec((R, NBLK, D), lambda i: (0, i, 0)),
            pl.BlockSpec((NBLK, D), lambda i: (i, 0)),
        ],
        out_shape=[
            jax.ShapeDtypeStruct((R, N, D), jnp.float32),
            jax.ShapeDtypeStruct((N, D), jnp.float32),
        ],
    )(y, stats, bng, bnb, w)


def _stage1_body(p_ref, cb_ref, w1_ref, b1_ref, hdn_ref, st_ref):
    i = pl.program_id(0)
    agg = p_ref[0] + p_ref[1] + cb_ref[...]
    hdn = jnp.dot(agg, w1_ref[...], preferred_element_type=jnp.float32)
    hdn = hdn + b1_ref[...]
    hdn_ref[...] = hdn
    s0 = jnp.sum(hdn, axis=0, keepdims=True)
    s1 = jnp.sum(hdn * hdn, axis=0, keepdims=True)
    st = jnp.concatenate([s0, s1], axis=0)

    @pl.when(i == 0)
    def _():
        st_ref[...] = jnp.zeros_like(st_ref)

    st_ref[...] += st


def _stage1(partials, cb, w1, b1):
    return pl.pallas_call(
        _stage1_body,
        grid=(NB,),
        in_specs=[
            pl.BlockSpec((NC, NBLK, D), lambda i: (0, i, 0)),
            pl.BlockSpec((D,), lambda i: (0,)),
            pl.BlockSpec((D, D), lambda i: (0, 0)),
            pl.BlockSpec((D,), lambda i: (0,)),
        ],
        out_specs=[
            pl.BlockSpec((NBLK, D), lambda i: (i, 0)),
            pl.BlockSpec((2, D), lambda i: (0, 0)),
        ],
        out_shape=[
            jax.ShapeDtypeStruct((N, D), jnp.float32),
            jax.ShapeDtypeStruct((2, D), jnp.float32),
        ],
    )(partials, cb, w1, b1)


def _stage2_body(hdn_ref, st_ref, g_ref, gb_ref, w2_ref, b2_ref,
                 y_ref, st2_ref):
    i = pl.program_id(0)
    mean = st_ref[0] * (1.0 / N)
    var = st_ref[1] * (1.0 / N) - mean * mean
    scale = lax.rsqrt(var + BN_EPS) * g_ref[...]
    shift = gb_ref[...] - mean * scale
    xb = jnp.maximum(hdn_ref[...] * scale + shift, 0.0)
    y = jnp.dot(xb, w2_ref[...], preferred_element_type=jnp.float32)
    y = y + b2_ref[...]
    y_ref[...] = y
    s0 = jnp.sum(y, axis=0, keepdims=True)
    s1 = jnp.sum(y * y, axis=0, keepdims=True)
    st = jnp.concatenate([s0, s1], axis=0)

    @pl.when(i == 0)
    def _():
        st2_ref[...] = jnp.zeros_like(st2_ref)

    st2_ref[...] += st


def _stage2(hdn, stats, g1, gb1, w2, b2):
    return pl.pallas_call(
        _stage2_body,
        grid=(NB,),
        in_specs=[
            pl.BlockSpec((NBLK, D), lambda i: (i, 0)),
            pl.BlockSpec((2, D), lambda i: (0, 0)),
            pl.BlockSpec((D,), lambda i: (0,)),
            pl.BlockSpec((D,), lambda i: (0,)),
            pl.BlockSpec((D, D), lambda i: (0, 0)),
            pl.BlockSpec((D,), lambda i: (0,)),
        ],
        out_specs=[
            pl.BlockSpec((NBLK, D), lambda i: (i, 0)),
            pl.BlockSpec((2, D), lambda i: (0, 0)),
        ],
        out_shape=[
            jax.ShapeDtypeStruct((N, D), jnp.float32),
            jax.ShapeDtypeStruct((2, D), jnp.float32),
        ],
    )(hdn, stats, g1, gb1, w2, b2)


def _stage3_body(y_ref, st_ref, g_ref, gb_ref, x_ref):
    mean = st_ref[0] * (1.0 / N)
    var = st_ref[1] * (1.0 / N) - mean * mean
    scale = lax.rsqrt(var + BN_EPS) * g_ref[...]
    shift = gb_ref[...] - mean * scale
    x_ref[...] = jnp.maximum(y_ref[...] * scale + shift, 0.0)


def _stage3(y, stats, bng, bnb):
    return pl.pallas_call(
        _stage3_body,
        grid=(NB,),
        in_specs=[
            pl.BlockSpec((NBLK, D), lambda i: (i, 0)),
            pl.BlockSpec((2, D), lambda i: (0, 0)),
            pl.BlockSpec((D,), lambda i: (0,)),
            pl.BlockSpec((D,), lambda i: (0,)),
        ],
        out_specs=pl.BlockSpec((NBLK, D), lambda i: (i, 0)),
        out_shape=jax.ShapeDtypeStruct((N, D), jnp.float32),
    )(y, stats, bng, bnb)


def _edge_agg_body(hall_ref, gidx_ref, dst_ref, out_ref,
                   gidx_v, dst_v, rows0, rows1, sem0, sem1, agg):
    c = lax.axis_index("c")
    s = lax.axis_index("s")
    wid = s * NC + c
    z = jnp.zeros((16,), jnp.float32)

    def zrow(j, carry):
        for k in range(D // 16):
            rows0[j, pl.ds(k * 16, 16)] = z
        return carry

    lax.fori_loop(0, CHUNK, zrow, 0)
    for t in range(ROWS_PT // CHUNK):
        pltpu.sync_copy(rows0, agg.at[pl.ds(s * ROWS_PT + t * CHUNK, CHUNK)])
    plsc.subcore_barrier()

    for p in range(NHALF):
        pltpu.sync_copy(gidx_ref.at[wid * NHALF + p], gidx_v)
        pltpu.sync_copy(dst_ref.at[wid * NHALF + p], dst_v)
        pltpu.async_copy(hall_ref.at[gidx_v.at[0]], rows0, sem0)

        def pair(i, carry):
            j0 = 2 * i
            pltpu.async_copy(hall_ref.at[gidx_v.at[j0 + 1]], rows1, sem1)
            pltpu.make_async_copy(hall_ref.at[gidx_v.at[j0]], rows0,
                                  sem0).wait()
            pltpu.sync_copy(rows0, agg.at[dst_v.at[j0]], add=True)

            @pl.when(j0 + 2 < HHALF)
            def _():
                pltpu.async_copy(hall_ref.at[gidx_v.at[j0 + 2]], rows0, sem0)

            pltpu.make_async_copy(hall_ref.at[gidx_v.at[j0 + 1]], rows1,
                                  sem1).wait()
            pltpu.sync_copy(rows1, agg.at[dst_v.at[j0 + 1]], add=True)
            return carry

        lax.fori_loop(0, HHALF // 2, pair, 0)
    plsc.subcore_barrier()
    pltpu.sync_copy(agg.at[pl.ds(s * ROWS_PT, ROWS_PT)],
                    out_ref.at[c, pl.ds(s * ROWS_PT, ROWS_PT)])


@functools.lru_cache(maxsize=None)
def _build_edge_agg():
    mesh = plsc.VectorSubcoreMesh(core_axis_name="c", subcore_axis_name="s")
    return pl.kernel(
        _edge_agg_body,
        out_type=jax.ShapeDtypeStruct((NC, NPAD, D), jnp.float32),
        name="edge_agg",
        mesh=mesh,
        scratch_types=[
            pltpu.VMEM((HHALF, CHUNK), jnp.int32),
            pltpu.VMEM((HHALF, CHUNK), jnp.int32),
            pltpu.VMEM((CHUNK, D), jnp.float32),
            pltpu.VMEM((CHUNK, D), jnp.float32),
            pltpu.SemaphoreType.DMA,
            pltpu.SemaphoreType.DMA,
            pltpu.VMEM_SHARED((NPAD, D), jnp.float32),
        ],
    )


def _mlp(hall, gidx, dst, cb, w1, b1, g1, gb1, w2, b2):
    partials = _build_edge_agg()(hall.reshape(R * N, D), gidx, dst)
    hdn, st1 = _stage1(partials, cb, w1, b1)
    return _stage2(hdn, st1, g1, gb1, w2, b2)


def kernel(h, edge_index, etypes,
           basis0, coeff0, cb0, w1_0, b1_0, g1_0, gb1_0, w2_0, b2_0,
           bng_0, bnb_0,
           basis1, coeff1, cb1, w1_1, b1_1, g1_1, gb1_1, w2_1, b2_1,
           bng_1, bnb_1):
    src, dst = edge_index[0], edge_index[1]
    gidx = etypes * N + src
    epw = E // NW
    padw = CPW * CHUNK - epw
    pad_g = jnp.tile(jnp.arange(padw, dtype=jnp.int32) * 331 % (R * N),
                     (NW, 1))
    pad_d = jnp.tile(N + jnp.arange(padw, dtype=jnp.int32) % (NPAD - N),
                     (NW, 1))
    gidx = jnp.concatenate([gidx.reshape(NW, epw), pad_g], axis=1)
    dstp = jnp.concatenate([dst.reshape(NW, epw), pad_d], axis=1)
    gidx = gidx.reshape(NW * NHALF, HHALF, CHUNK)
    dstp = dstp.reshape(NW * NHALF, HHALF, CHUNK)

    hall1 = _hall(h, _wcomb(coeff0, basis0))
    y1, sty1 = _mlp(hall1, gidx, dstp, cb0, w1_0, b1_0,
                    g1_0, gb1_0, w2_0, b2_0)
    hall2, x1 = _hall_bn(y1, sty1, bng_0, bnb_0, _wcomb(coeff1, basis1))
    y2, sty2 = _mlp(hall2, gidx, dstp, cb1, w1_1, b1_1,
                    g1_1, gb1_1, w2_1, b2_1)
    x2 = _stage3(y2, sty2, bng_1, bnb_1)
    return jnp.stack([h, x1, x2])


# speedup vs baseline: 1.2839x; 1.0355x over previous
import functools

import jax
import jax.numpy as jnp
from jax import lax
from jax.experimental import pallas as pl
from jax.experimental.pallas import tpu as pltpu
from jax.experimental.pallas import tpu_sc as plsc

N, E, D, R, B_ = 10000, 320000, 128, 8, 8
NC, NS = 2, 16
NW = NC * NS
CHUNK = 128
CPW = 80
HHALF = CPW // 2
NHALF = CPW // HHALF
EPAD = NW * CPW * CHUNK
NPAD = 10240
ROWS_PT = NPAD // NS
BN_EPS = 1e-5
NBLK = 2000
NB = N // NBLK


def _wcomb_body(coeff_ref, basis_ref, w_ref):
    w_ref[...] = jnp.dot(coeff_ref[...], basis_ref[...],
                         preferred_element_type=jnp.float32)


def _wcomb(coeff, basis):
    w = pl.pallas_call(
        _wcomb_body,
        out_shape=jax.ShapeDtypeStruct((R, D * D), jnp.float32),
    )(coeff, basis.reshape(B_, D * D))
    return w.reshape(R, D, D)


def _hall_body(x_ref, w_ref, out_ref):
    x = x_ref[...]
    for r in range(R):
        out_ref[r] = jnp.dot(x, w_ref[r], preferred_element_type=jnp.float32)


def _hall(x, w):
    return pl.pallas_call(
        _hall_body,
        grid=(NB,),
        in_specs=[
            pl.BlockSpec((NBLK, D), lambda i: (i, 0)),
            pl.BlockSpec((R, D, D), lambda i: (0, 0, 0)),
        ],
        out_specs=pl.BlockSpec((R, NBLK, D), lambda i: (0, i, 0)),
        out_shape=jax.ShapeDtypeStruct((R, N, D), jnp.float32),
    )(x, w)


def _hall_bn_body(y_ref, st_ref, g_ref, gb_ref, w_ref, out_ref, x_ref):
    mean = st_ref[0] * (1.0 / N)
    var = st_ref[1] * (1.0 / N) - mean * mean
    scale = lax.rsqrt(var + BN_EPS) * g_ref[...]
    shift = gb_ref[...] - mean * scale
    xb = jnp.maximum(y_ref[...] * scale + shift, 0.0)
    x_ref[...] = xb
    for r in range(R):
        out_ref[r] = jnp.dot(xb, w_ref[r], preferred_element_type=jnp.float32)


def _hall_bn(y, stats, bng, bnb, w):
    return pl.pallas_call(
        _hall_bn_body,
        grid=(NB,),
        in_specs=[
            pl.BlockSpec((NBLK, D), lambda i: (i, 0)),
            pl.BlockSpec((2, D), lambda i: (0, 0)),
            pl.BlockSpec((D,), lambda i: (0,)),
            pl.BlockSpec((D,), lambda i: (0,)),
            pl.BlockSpec((R, D, D), lambda i: (0, 0, 0)),
        ],
        out_specs=[
            pl.BlockSpec((R, NBLK, D), lambda i: (0, i, 0)),
            pl.BlockSpec((NBLK, D), lambda i: (i, 0)),
        ],
        out_shape=[
            jax.ShapeDtypeStruct((R, N, D), jnp.float32),
            jax.ShapeDtypeStruct((N, D), jnp.float32),
        ],
    )(y, stats, bng, bnb, w)


def _stage1_body(p_ref, cb_ref, w1_ref, b1_ref, hdn_ref, st_ref):
    i = pl.program_id(0)
    agg = p_ref[0] + p_ref[1] + cb_ref[...]
    hdn = jnp.dot(agg, w1_ref[...], preferred_element_type=jnp.float32)
    hdn = hdn + b1_ref[...]
    hdn_ref[...] = hdn
    s0 = jnp.sum(hdn, axis=0, keepdims=True)
    s1 = jnp.sum(hdn * hdn, axis=0, keepdims=True)
    st = jnp.concatenate([s0, s1], axis=0)

    @pl.when(i == 0)
    def _():
        st_ref[...] = jnp.zeros_like(st_ref)

    st_ref[...] += st


def _stage1(partials, cb, w1, b1):
    return pl.pallas_call(
        _stage1_body,
        grid=(NB,),
        in_specs=[
            pl.BlockSpec((NC, NBLK, D), lambda i: (0, i, 0)),
            pl.BlockSpec((D,), lambda i: (0,)),
            pl.BlockSpec((D, D), lambda i: (0, 0)),
            pl.BlockSpec((D,), lambda i: (0,)),
        ],
        out_specs=[
            pl.BlockSpec((NBLK, D), lambda i: (i, 0)),
            pl.BlockSpec((2, D), lambda i: (0, 0)),
        ],
        out_shape=[
            jax.ShapeDtypeStruct((N, D), jnp.float32),
            jax.ShapeDtypeStruct((2, D), jnp.float32),
        ],
    )(partials, cb, w1, b1)


def _stage2_body(hdn_ref, st_ref, g_ref, gb_ref, w2_ref, b2_ref,
                 y_ref, st2_ref):
    i = pl.program_id(0)
    mean = st_ref[0] * (1.0 / N)
    var = st_ref[1] * (1.0 / N) - mean * mean
    scale = lax.rsqrt(var + BN_EPS) * g_ref[...]
    shift = gb_ref[...] - mean * scale
    xb = jnp.maximum(hdn_ref[...] * scale + shift, 0.0)
    y = jnp.dot(xb, w2_ref[...], preferred_element_type=jnp.float32)
    y = y + b2_ref[...]
    y_ref[...] = y
    s0 = jnp.sum(y, axis=0, keepdims=True)
    s1 = jnp.sum(y * y, axis=0, keepdims=True)
    st = jnp.concatenate([s0, s1], axis=0)

    @pl.when(i == 0)
    def _():
        st2_ref[...] = jnp.zeros_like(st2_ref)

    st2_ref[...] += st


def _stage2(hdn, stats, g1, gb1, w2, b2):
    return pl.pallas_call(
        _stage2_body,
        grid=(NB,),
        in_specs=[
            pl.BlockSpec((NBLK, D), lambda i: (i, 0)),
            pl.BlockSpec((2, D), lambda i: (0, 0)),
            pl.BlockSpec((D,), lambda i: (0,)),
            pl.BlockSpec((D,), lambda i: (0,)),
            pl.BlockSpec((D, D), lambda i: (0, 0)),
            pl.BlockSpec((D,), lambda i: (0,)),
        ],
        out_specs=[
            pl.BlockSpec((NBLK, D), lambda i: (i, 0)),
            pl.BlockSpec((2, D), lambda i: (0, 0)),
        ],
        out_shape=[
            jax.ShapeDtypeStruct((N, D), jnp.float32),
            jax.ShapeDtypeStruct((2, D), jnp.float32),
        ],
    )(hdn, stats, g1, gb1, w2, b2)


def _stage3_body(h_ref, x1_ref, y_ref, st_ref, g_ref, gb_ref, out_ref):
    mean = st_ref[0] * (1.0 / N)
    var = st_ref[1] * (1.0 / N) - mean * mean
    scale = lax.rsqrt(var + BN_EPS) * g_ref[...]
    shift = gb_ref[...] - mean * scale
    out_ref[0] = h_ref[...]
    out_ref[1] = x1_ref[...]
    out_ref[2] = jnp.maximum(y_ref[...] * scale + shift, 0.0)


def _stage3(h, x1, y, stats, bng, bnb):
    return pl.pallas_call(
        _stage3_body,
        grid=(NB,),
        in_specs=[
            pl.BlockSpec((NBLK, D), lambda i: (i, 0)),
            pl.BlockSpec((NBLK, D), lambda i: (i, 0)),
            pl.BlockSpec((NBLK, D), lambda i: (i, 0)),
            pl.BlockSpec((2, D), lambda i: (0, 0)),
            pl.BlockSpec((D,), lambda i: (0,)),
            pl.BlockSpec((D,), lambda i: (0,)),
        ],
        out_specs=pl.BlockSpec((3, NBLK, D), lambda i: (0, i, 0)),
        out_shape=jax.ShapeDtypeStruct((3, N, D), jnp.float32),
    )(h, x1, y, stats, bng, bnb)


def _edge_agg_body(hall_ref, key_ref, out_ref,
                   gidx_v, dst_v, rows0, rows1, sem0, sem1, agg):
    c = lax.axis_index("c")
    s = lax.axis_index("s")
    wid = s * NC + c
    z = jnp.zeros((16,), jnp.float32)

    def zrow(j, carry):
        for k in range(D // 16):
            rows0[j, pl.ds(k * 16, 16)] = z
        return carry

    lax.fori_loop(0, CHUNK, zrow, 0)
    for t in range(ROWS_PT // CHUNK):
        pltpu.sync_copy(rows0, agg.at[pl.ds(s * ROWS_PT + t * CHUNK, CHUNK)])
    plsc.subcore_barrier()

    for p in range(NHALF):
        pltpu.sync_copy(key_ref.at[wid * NHALF + p], gidx_v)

        def unpack(j, carry):
            for k in range(CHUNK // 16):
                kv = gidx_v[j, pl.ds(k * 16, 16)]
                gidx_v[j, pl.ds(k * 16, 16)] = kv >> 14
                dst_v[j, pl.ds(k * 16, 16)] = kv & 16383
            return carry

        lax.fori_loop(0, HHALF, unpack, 0)
        pltpu.async_copy(hall_ref.at[gidx_v.at[0]], rows0, sem0)

        def pair(i, carry):
            j0 = 2 * i
            pltpu.async_copy(hall_ref.at[gidx_v.at[j0 + 1]], rows1, sem1)
            pltpu.make_async_copy(hall_ref.at[gidx_v.at[j0]], rows0,
                                  sem0).wait()
            pltpu.sync_copy(rows0, agg.at[dst_v.at[j0]], add=True)

            @pl.when(j0 + 2 < HHALF)
            def _():
                pltpu.async_copy(hall_ref.at[gidx_v.at[j0 + 2]], rows0, sem0)

            pltpu.make_async_copy(hall_ref.at[gidx_v.at[j0 + 1]], rows1,
                                  sem1).wait()
            pltpu.sync_copy(rows1, agg.at[dst_v.at[j0 + 1]], add=True)
            return carry

        lax.fori_loop(0, HHALF // 2, pair, 0)
    plsc.subcore_barrier()
    pltpu.sync_copy(agg.at[pl.ds(s * ROWS_PT, ROWS_PT)],
                    out_ref.at[c, pl.ds(s * ROWS_PT, ROWS_PT)])


@functools.lru_cache(maxsize=None)
def _build_edge_agg():
    mesh = plsc.VectorSubcoreMesh(core_axis_name="c", subcore_axis_name="s")
    return pl.kernel(
        _edge_agg_body,
        out_type=jax.ShapeDtypeStruct((NC, NPAD, D), jnp.float32),
        name="edge_agg",
        mesh=mesh,
        scratch_types=[
            pltpu.VMEM((HHALF, CHUNK), jnp.int32),
            pltpu.VMEM((HHALF, CHUNK), jnp.int32),
            pltpu.VMEM((CHUNK, D), jnp.float32),
            pltpu.VMEM((CHUNK, D), jnp.float32),
            pltpu.SemaphoreType.DMA,
            pltpu.SemaphoreType.DMA,
            pltpu.VMEM_SHARED((NPAD, D), jnp.float32),
        ],
    )


def _mlp(hall, keys, cb, w1, b1, g1, gb1, w2, b2):
    partials = _build_edge_agg()(hall.reshape(R * N, D), keys)
    hdn, st1 = _stage1(partials, cb, w1, b1)
    return _stage2(hdn, st1, g1, gb1, w2, b2)


def kernel(h, edge_index, etypes,
           basis0, coeff0, cb0, w1_0, b1_0, g1_0, gb1_0, w2_0, b2_0,
           bng_0, bnb_0,
           basis1, coeff1, cb1, w1_1, b1_1, g1_1, gb1_1, w2_1, b2_1,
           bng_1, bnb_1):
    src, dst = edge_index[0], edge_index[1]
    key = (etypes * N + src) * 16384 + dst
    epw = E // NW
    padw = CPW * CHUNK - epw
    pidx = jnp.arange(padw, dtype=jnp.int32)
    pad_k = jnp.tile((pidx * 331 % (R * N)) * 16384
                     + N + pidx % (NPAD - N), (NW, 1))
    keys = jnp.concatenate([key.reshape(NW, epw), pad_k], axis=1)
    keys = keys.reshape(NW * NHALF, HHALF, CHUNK)

    hall1 = _hall(h, _wcomb(coeff0, basis0))
    y1, sty1 = _mlp(hall1, keys, cb0, w1_0, b1_0,
                    g1_0, gb1_0, w2_0, b2_0)
    hall2, x1 = _hall_bn(y1, sty1, bng_0, bnb_0, _wcomb(coeff1, basis1))
    y2, sty2 = _mlp(hall2, keys, cb1, w1_1, b1_1,
                    g1_1, gb1_1, w2_1, b2_1)
    return _stage3(h, x1, y2, sty2, bng_1, bnb_1)


# speedup vs baseline: 1.3207x; 1.0286x over previous
import functools

import jax
import jax.numpy as jnp
from jax import lax
from jax.experimental import pallas as pl
from jax.experimental.pallas import tpu as pltpu
from jax.experimental.pallas import tpu_sc as plsc

N, E, D, R, B_ = 10000, 320000, 128, 8, 8
NC, NS = 2, 16
NW = NC * NS
CHUNK = 64
CPW = 160
NBUF = 4
HHALF = CPW // 4
NHALF = CPW // HHALF
EPAD = NW * CPW * CHUNK
NPAD = 10240
ROWS_PT = NPAD // NS
BN_EPS = 1e-5
NBLK = 2000
NB = N // NBLK


def _wcomb_body(coeff_ref, basis_ref, w_ref):
    w_ref[...] = jnp.dot(coeff_ref[...], basis_ref[...],
                         preferred_element_type=jnp.float32)


def _wcomb(coeff, basis):
    w = pl.pallas_call(
        _wcomb_body,
        out_shape=jax.ShapeDtypeStruct((R, D * D), jnp.float32),
    )(coeff, basis.reshape(B_, D * D))
    return w.reshape(R, D, D)


def _hall_body(x_ref, w_ref, out_ref):
    x = x_ref[...]
    for r in range(R):
        out_ref[r] = jnp.dot(x, w_ref[r], preferred_element_type=jnp.float32)


def _hall(x, w):
    return pl.pallas_call(
        _hall_body,
        grid=(NB,),
        in_specs=[
            pl.BlockSpec((NBLK, D), lambda i: (i, 0)),
            pl.BlockSpec((R, D, D), lambda i: (0, 0, 0)),
        ],
        out_specs=pl.BlockSpec((R, NBLK, D), lambda i: (0, i, 0)),
        out_shape=jax.ShapeDtypeStruct((R, N, D), jnp.float32),
    )(x, w)


def _hall_bn_body(y_ref, st_ref, g_ref, gb_ref, w_ref, out_ref, x_ref):
    mean = st_ref[0] * (1.0 / N)
    var = st_ref[1] * (1.0 / N) - mean * mean
    scale = lax.rsqrt(var + BN_EPS) * g_ref[...]
    shift = gb_ref[...] - mean * scale
    xb = jnp.maximum(y_ref[...] * scale + shift, 0.0)
    x_ref[...] = xb
    for r in range(R):
        out_ref[r] = jnp.dot(xb, w_ref[r], preferred_element_type=jnp.float32)


def _hall_bn(y, stats, bng, bnb, w):
    return pl.pallas_call(
        _hall_bn_body,
        grid=(NB,),
        in_specs=[
            pl.BlockSpec((NBLK, D), lambda i: (i, 0)),
            pl.BlockSpec((2, D), lambda i: (0, 0)),
            pl.BlockSpec((D,), lambda i: (0,)),
            pl.BlockSpec((D,), lambda i: (0,)),
            pl.BlockSpec((R, D, D), lambda i: (0, 0, 0)),
        ],
        out_specs=[
            pl.BlockSpec((R, NBLK, D), lambda i: (0, i, 0)),
            pl.BlockSpec((NBLK, D), lambda i: (i, 0)),
        ],
        out_shape=[
            jax.ShapeDtypeStruct((R, N, D), jnp.float32),
            jax.ShapeDtypeStruct((N, D), jnp.float32),
        ],
    )(y, stats, bng, bnb, w)


def _stage1_body(p_ref, cb_ref, w1_ref, b1_ref, hdn_ref, st_ref):
    i = pl.program_id(0)
    agg = p_ref[0] + p_ref[1] + cb_ref[...]
    hdn = jnp.dot(agg, w1_ref[...], preferred_element_type=jnp.float32)
    hdn = hdn + b1_ref[...]
    hdn_ref[...] = hdn
    s0 = jnp.sum(hdn, axis=0, keepdims=True)
    s1 = jnp.sum(hdn * hdn, axis=0, keepdims=True)
    st = jnp.concatenate([s0, s1], axis=0)

    @pl.when(i == 0)
    def _():
        st_ref[...] = jnp.zeros_like(st_ref)

    st_ref[...] += st


def _stage1(partials, cb, w1, b1):
    return pl.pallas_call(
        _stage1_body,
        grid=(NB,),
        in_specs=[
            pl.BlockSpec((NC, NBLK, D), lambda i: (0, i, 0)),
            pl.BlockSpec((D,), lambda i: (0,)),
            pl.BlockSpec((D, D), lambda i: (0, 0)),
            pl.BlockSpec((D,), lambda i: (0,)),
        ],
        out_specs=[
            pl.BlockSpec((NBLK, D), lambda i: (i, 0)),
            pl.BlockSpec((2, D), lambda i: (0, 0)),
        ],
        out_shape=[
            jax.ShapeDtypeStruct((N, D), jnp.float32),
            jax.ShapeDtypeStruct((2, D), jnp.float32),
        ],
    )(partials, cb, w1, b1)


def _stage2_body(hdn_ref, st_ref, g_ref, gb_ref, w2_ref, b2_ref,
                 y_ref, st2_ref):
    i = pl.program_id(0)
    mean = st_ref[0] * (1.0 / N)
    var = st_ref[1] * (1.0 / N) - mean * mean
    scale = lax.rsqrt(var + BN_EPS) * g_ref[...]
    shift = gb_ref[...] - mean * scale
    xb = jnp.maximum(hdn_ref[...] * scale + shift, 0.0)
    y = jnp.dot(xb, w2_ref[...], preferred_element_type=jnp.float32)
    y = y + b2_ref[...]
    y_ref[...] = y
    s0 = jnp.sum(y, axis=0, keepdims=True)
    s1 = jnp.sum(y * y, axis=0, keepdims=True)
    st = jnp.concatenate([s0, s1], axis=0)

    @pl.when(i == 0)
    def _():
        st2_ref[...] = jnp.zeros_like(st2_ref)

    st2_ref[...] += st


def _stage2(hdn, stats, g1, gb1, w2, b2):
    return pl.pallas_call(
        _stage2_body,
        grid=(NB,),
        in_specs=[
            pl.BlockSpec((NBLK, D), lambda i: (i, 0)),
            pl.BlockSpec((2, D), lambda i: (0, 0)),
            pl.BlockSpec((D,), lambda i: (0,)),
            pl.BlockSpec((D,), lambda i: (0,)),
            pl.BlockSpec((D, D), lambda i: (0, 0)),
            pl.BlockSpec((D,), lambda i: (0,)),
        ],
        out_specs=[
            pl.BlockSpec((NBLK, D), lambda i: (i, 0)),
            pl.BlockSpec((2, D), lambda i: (0, 0)),
        ],
        out_shape=[
            jax.ShapeDtypeStruct((N, D), jnp.float32),
            jax.ShapeDtypeStruct((2, D), jnp.float32),
        ],
    )(hdn, stats, g1, gb1, w2, b2)


def _stage3_body(h_ref, x1_ref, y_ref, st_ref, g_ref, gb_ref, out_ref):
    mean = st_ref[0] * (1.0 / N)
    var = st_ref[1] * (1.0 / N) - mean * mean
    scale = lax.rsqrt(var + BN_EPS) * g_ref[...]
    shift = gb_ref[...] - mean * scale
    out_ref[0] = h_ref[...]
    out_ref[1] = x1_ref[...]
    out_ref[2] = jnp.maximum(y_ref[...] * scale + shift, 0.0)


def _stage3(h, x1, y, stats, bng, bnb):
    return pl.pallas_call(
        _stage3_body,
        grid=(NB,),
        in_specs=[
            pl.BlockSpec((NBLK, D), lambda i: (i, 0)),
            pl.BlockSpec((NBLK, D), lambda i: (i, 0)),
            pl.BlockSpec((NBLK, D), lambda i: (i, 0)),
            pl.BlockSpec((2, D), lambda i: (0, 0)),
            pl.BlockSpec((D,), lambda i: (0,)),
            pl.BlockSpec((D,), lambda i: (0,)),
        ],
        out_specs=pl.BlockSpec((3, NBLK, D), lambda i: (0, i, 0)),
        out_shape=jax.ShapeDtypeStruct((3, N, D), jnp.float32),
    )(h, x1, y, stats, bng, bnb)


def _edge_agg_body(hall_ref, key_ref, out_ref,
                   gidx_v, dst_v, rows0, rows1, rows2, rows3,
                   sem0, sem1, sem2, sem3, agg):
    rows = (rows0, rows1, rows2, rows3)
    sems = (sem0, sem1, sem2, sem3)
    c = lax.axis_index("c")
    s = lax.axis_index("s")
    wid = s * NC + c
    z = jnp.zeros((16,), jnp.float32)

    def zrow(j, carry):
        for k in range(D // 16):
            rows0[j, pl.ds(k * 16, 16)] = z
        return carry

    lax.fori_loop(0, CHUNK, zrow, 0)
    for t in range(ROWS_PT // CHUNK):
        pltpu.sync_copy(rows0, agg.at[pl.ds(s * ROWS_PT + t * CHUNK, CHUNK)])
    plsc.subcore_barrier()

    for p in range(NHALF):
        pltpu.sync_copy(key_ref.at[wid * NHALF + p], gidx_v)

        def unpack(j, carry):
            for k in range(CHUNK // 16):
                kv = gidx_v[j, pl.ds(k * 16, 16)]
                gidx_v[j, pl.ds(k * 16, 16)] = kv >> 14
                dst_v[j, pl.ds(k * 16, 16)] = kv & 16383
            return carry

        lax.fori_loop(0, HHALF, unpack, 0)
        for b in range(NBUF):
            pltpu.async_copy(hall_ref.at[gidx_v.at[b]], rows[b], sems[b])

        def ring(i, carry):
            j0 = NBUF * i
            for b in range(NBUF):
                j = j0 + b
                pltpu.make_async_copy(hall_ref.at[gidx_v.at[j]], rows[b],
                                      sems[b]).wait()
                pltpu.sync_copy(rows[b], agg.at[dst_v.at[j]], add=True)

                @pl.when(j + NBUF < HHALF)
                def _():
                    pltpu.async_copy(hall_ref.at[gidx_v.at[j + NBUF]],
                                     rows[b], sems[b])
            return carry

        lax.fori_loop(0, HHALF // NBUF, ring, 0)
    plsc.subcore_barrier()
    pltpu.sync_copy(agg.at[pl.ds(s * ROWS_PT, ROWS_PT)],
                    out_ref.at[c, pl.ds(s * ROWS_PT, ROWS_PT)])


@functools.lru_cache(maxsize=None)
def _build_edge_agg():
    mesh = plsc.VectorSubcoreMesh(core_axis_name="c", subcore_axis_name="s")
    return pl.kernel(
        _edge_agg_body,
        out_type=jax.ShapeDtypeStruct((NC, NPAD, D), jnp.float32),
        name="edge_agg",
        mesh=mesh,
        scratch_types=[
            pltpu.VMEM((HHALF, CHUNK), jnp.int32),
            pltpu.VMEM((HHALF, CHUNK), jnp.int32),
            pltpu.VMEM((CHUNK, D), jnp.float32),
            pltpu.VMEM((CHUNK, D), jnp.float32),
            pltpu.VMEM((CHUNK, D), jnp.float32),
            pltpu.VMEM((CHUNK, D), jnp.float32),
            pltpu.SemaphoreType.DMA,
            pltpu.SemaphoreType.DMA,
            pltpu.SemaphoreType.DMA,
            pltpu.SemaphoreType.DMA,
            pltpu.VMEM_SHARED((NPAD, D), jnp.float32),
        ],
    )


def _mlp(hall, keys, cb, w1, b1, g1, gb1, w2, b2):
    partials = _build_edge_agg()(hall.reshape(R * N, D), keys)
    hdn, st1 = _stage1(partials, cb, w1, b1)
    return _stage2(hdn, st1, g1, gb1, w2, b2)


def kernel(h, edge_index, etypes,
           basis0, coeff0, cb0, w1_0, b1_0, g1_0, gb1_0, w2_0, b2_0,
           bng_0, bnb_0,
           basis1, coeff1, cb1, w1_1, b1_1, g1_1, gb1_1, w2_1, b2_1,
           bng_1, bnb_1):
    src, dst = edge_index[0], edge_index[1]
    key = (etypes * N + src) * 16384 + dst
    epw = E // NW
    padw = CPW * CHUNK - epw
    pidx = jnp.arange(padw, dtype=jnp.int32)
    pad_k = jnp.tile((pidx * 331 % (R * N)) * 16384
                     + N + pidx % (NPAD - N), (NW, 1))
    keys = jnp.concatenate([key.reshape(NW, epw), pad_k], axis=1)
    keys = keys.reshape(NW * NHALF, HHALF, CHUNK)

    hall1 = _hall(h, _wcomb(coeff0, basis0))
    y1, sty1 = _mlp(hall1, keys, cb0, w1_0, b1_0,
                    g1_0, gb1_0, w2_0, b2_0)
    hall2, x1 = _hall_bn(y1, sty1, bng_0, bnb_0, _wcomb(coeff1, basis1))
    y2, sty2 = _mlp(hall2, keys, cb1, w1_1, b1_1,
                    g1_1, gb1_1, w2_1, b2_1)
    return _stage3(h, x1, y2, sty2, bng_1, bnb_1)
